# baseline (speedup 1.0000x reference)
import functools

import jax
import jax.numpy as jnp
from jax import lax
from jax.experimental import pallas as pl
from jax.experimental.pallas import tpu as pltpu
from jax.experimental.pallas import tpu_sc as plsc

_NUM_CLASSES = 1000000
_FEAT_DIM = 64
_BATCH = 16384
_LAMBDA_C = 0.001

_INFO = plsc.get_sparse_core_info()
_NC, _NS, _L = _INFO.num_cores, _INFO.num_subcores, _INFO.num_lanes
_NW = _NC * _NS
_ROWS_PER_W = _BATCH // _NW
_IDX_CHUNK = 128
_N_CHUNKS = _ROWS_PER_W // _IDX_CHUNK


def _sc_body(feats_hbm, lab_hbm, centers_hbm, out_hbm,
             idx_v, feats_v, rows_v, acc_v, gsem, fsem):
    wid = lax.axis_index("s") * _NC + lax.axis_index("c")
    base = wid * _ROWS_PER_W

    pltpu.sync_copy(lab_hbm.at[wid], idx_v)
    fcopy = pltpu.async_copy(
        feats_hbm.at[pl.ds(base, _ROWS_PER_W)], feats_v, fsem)
    gathers = [
        pltpu.async_copy(
            centers_hbm.at[idx_v.at[j]],
            rows_v.at[pl.ds(j * _IDX_CHUNK, _IDX_CHUNK)],
            gsem,
        )
        for j in range(_N_CHUNKS)
    ]
    fcopy.wait()
    for g in gathers:
        g.wait()

    def body(r, acc):
        for c in range(_FEAT_DIM // _L):
            f = feats_v[r, pl.ds(c * _L, _L)]
            g = rows_v[r, pl.ds(c * _L, _L)]
            d = f - g
            acc = acc + d * d
        return acc

    acc = lax.fori_loop(0, _ROWS_PER_W, body, jnp.zeros((_L,), jnp.float32))
    acc_v[...] = acc
    pltpu.sync_copy(acc_v, out_hbm.at[wid])


@jax.jit
def kernel(features, labels, centers):
    labels = labels.astype(jnp.int32).reshape(_NW, _N_CHUNKS, _IDX_CHUNK)
    mesh = plsc.VectorSubcoreMesh(core_axis_name="c", subcore_axis_name="s")
    partials = pl.kernel(
        _sc_body,
        out_type=jax.ShapeDtypeStruct((_NW, _L), jnp.float32),
        mesh=mesh,
        scratch_types=[
            pltpu.VMEM((_N_CHUNKS, _IDX_CHUNK), jnp.int32),
            pltpu.VMEM((_ROWS_PER_W, _FEAT_DIM), jnp.float32),
            pltpu.VMEM((_ROWS_PER_W, _FEAT_DIM), jnp.float32),
            pltpu.VMEM((_L,), jnp.float32),
            pltpu.SemaphoreType.DMA,
            pltpu.SemaphoreType.DMA,
        ],
        compiler_params=pltpu.CompilerParams(use_tc_tiling_on_sc=False),
    )(features, labels, centers)
    return _LAMBDA_C * jnp.sum(partials) / 2.0 / _BATCH


# speedup vs baseline: 1.6668x; 1.6668x over previous
import jax
import jax.numpy as jnp
from jax import lax
from jax.experimental import pallas as pl
from jax.experimental.pallas import tpu as pltpu
from jax.experimental.pallas import tpu_sc as plsc

_NUM_CLASSES = 1000000
_FEAT_DIM = 64
_BATCH = 16384
_LAMBDA_C = 0.001

_INFO = plsc.get_sparse_core_info()
_NC, _NS, _L = _INFO.num_cores, _INFO.num_subcores, _INFO.num_lanes
_NW = _NC * _NS
_ROWS_PER_W = _BATCH // _NW
_CHUNK = 32
_N_CHUNKS = _ROWS_PER_W // _CHUNK


def _sc_body(feats_hbm, lab_hbm, centers_hbm, out_hbm,
             lab_v, rows_v, fchunk_v, acc_v, sems):
    wid = lax.axis_index("s") * _NC + lax.axis_index("c")
    base = wid * _ROWS_PER_W

    pltpu.sync_copy(lab_hbm.at[pl.ds(base, _ROWS_PER_W)], lab_v)

    def issue(ch, buf):
        def fire_group(g, _):
            off = pl.multiple_of(ch * _CHUNK + g * _L, _L)
            vec = lab_v[pl.ds(off, _L)]
            for l in range(_L):
                pltpu.async_copy(
                    centers_hbm.at[vec[l]],
                    rows_v.at[buf, g * _L + l],
                    sems.at[2 * buf])
            return 0

        lax.fori_loop(0, _CHUNK // _L, fire_group, 0)
        pltpu.async_copy(
            feats_hbm.at[pl.ds(base + ch * _CHUNK, _CHUNK)],
            fchunk_v.at[buf], sems.at[2 * buf + 1])

    def drain(buf):
        pltpu.make_async_copy(
            centers_hbm.at[pl.ds(0, _CHUNK)], rows_v.at[buf],
            sems.at[2 * buf]).wait()
        pltpu.make_async_copy(
            feats_hbm.at[pl.ds(0, _CHUNK)], fchunk_v.at[buf],
            sems.at[2 * buf + 1]).wait()

    issue(0, 0)
    issue(1, 1)
    acc = jnp.zeros((_L,), jnp.float32)

    for ch in range(_N_CHUNKS):
        buf = ch % 2
        drain(buf)

        def label_body(i, acc, buf=buf):
            for c in range(_FEAT_DIM // _L):
                f = fchunk_v[buf, i, pl.ds(c * _L, _L)]
                g = rows_v[buf, i, pl.ds(c * _L, _L)]
                d = f - g
                acc = acc + d * d
            return acc

        acc = lax.fori_loop(0, _CHUNK, label_body, acc)
        if ch + 2 < _N_CHUNKS:
            issue(ch + 2, buf)

    acc_v[...] = acc
    pltpu.sync_copy(acc_v, out_hbm.at[pl.ds(wid * _L, _L)])


@jax.jit
def kernel(features, labels, centers):
    labels = labels.astype(jnp.int32)
    mesh = plsc.VectorSubcoreMesh(core_axis_name="c", subcore_axis_name="s")
    partials = pl.kernel(
        _sc_body,
        out_type=jax.ShapeDtypeStruct((_NW * _L,), jnp.float32),
        mesh=mesh,
        scratch_types=[
            pltpu.VMEM((_ROWS_PER_W,), jnp.int32),
            pltpu.VMEM((2, _CHUNK, _FEAT_DIM), jnp.float32),
            pltpu.VMEM((2, _CHUNK, _FEAT_DIM), jnp.float32),
            pltpu.VMEM((_L,), jnp.float32),
            pltpu.SemaphoreType.DMA((4,)),
        ],
        compiler_params=pltpu.CompilerParams(needs_layout_passes=False),
    )(features, labels, centers)
    return _LAMBDA_C * jnp.sum(partials) / 2.0 / _BATCH


# speedup vs baseline: 1.6702x; 1.0020x over previous
import jax
import jax.numpy as jnp
from jax import lax
from jax.experimental import pallas as pl
from jax.experimental.pallas import tpu as pltpu
from jax.experimental.pallas import tpu_sc as plsc

_NUM_CLASSES = 1000000
_FEAT_DIM = 64
_BATCH = 16384
_LAMBDA_C = 0.001

_INFO = plsc.get_sparse_core_info()
_NC, _NS, _L = _INFO.num_cores, _INFO.num_subcores, _INFO.num_lanes
_NW = _NC * _NS
_ROWS_PER_W = _BATCH // _NW
_CHUNK = 32
_N_CHUNKS = _ROWS_PER_W // _CHUNK


def _sc_body(feats_hbm, lab_hbm, centers_hbm, out_hbm,
             lab_v, rows_v, fchunk_v, acc_v, sems):
    wid = lax.axis_index("s") * _NC + lax.axis_index("c")
    base = wid * _ROWS_PER_W

    pltpu.sync_copy(lab_hbm.at[pl.ds(base, _ROWS_PER_W)], lab_v)

    def issue(ch, buf):
        def fire_group(g, _):
            off = pl.multiple_of(ch * _CHUNK + g * _L, _L)
            vec = lab_v[pl.ds(off, _L)]
            for l in range(_L):
                pltpu.async_copy(
                    centers_hbm.at[vec[l]],
                    rows_v.at[buf, g * _L + l],
                    sems.at[2 * buf])
            return 0

        lax.fori_loop(0, _CHUNK // _L, fire_group, 0)
        pltpu.async_copy(
            feats_hbm.at[pl.ds(base + ch * _CHUNK, _CHUNK)],
            fchunk_v.at[buf], sems.at[2 * buf + 1])

    def drain(buf):
        pltpu.make_async_copy(
            centers_hbm.at[pl.ds(0, _CHUNK)], rows_v.at[buf],
            sems.at[2 * buf]).wait()
        pltpu.make_async_copy(
            feats_hbm.at[pl.ds(0, _CHUNK)], fchunk_v.at[buf],
            sems.at[2 * buf + 1]).wait()

    issue(0, 0)
    issue(1, 1)
    acc = jnp.zeros((_L,), jnp.float32)

    for ch in range(_N_CHUNKS):
        buf = ch % 2
        drain(buf)

        def label_body(i, acc, buf=buf):
            for c in range(_FEAT_DIM // _L):
                f = fchunk_v[buf, i, pl.ds(c * _L, _L)]
                g = rows_v[buf, i, pl.ds(c * _L, _L)]
                d = f - g
                acc = acc + d * d
            return acc

        acc = lax.fori_loop(0, _CHUNK, label_body, acc)
        if ch + 2 < _N_CHUNKS:
            issue(ch + 2, buf)

    acc_v[...] = acc
    pltpu.sync_copy(acc_v, out_hbm.at[pl.ds(wid * _L, _L)])


@jax.jit
def kernel(features, labels, centers):
    labels = labels.astype(jnp.int32)
    mesh = plsc.VectorSubcoreMesh(core_axis_name="c", subcore_axis_name="s")
    partials = pl.kernel(
        _sc_body,
        out_type=jax.ShapeDtypeStruct((_NW * _L,), jnp.float32),
        mesh=mesh,
        scratch_types=[
            pltpu.VMEM((_ROWS_PER_W,), jnp.int32),
            pltpu.VMEM((2, _CHUNK, _FEAT_DIM), jnp.float32),
            pltpu.VMEM((2, _CHUNK, _FEAT_DIM), jnp.float32),
            pltpu.VMEM((_L,), jnp.float32),
            pltpu.SemaphoreType.DMA((4,)),
        ],
    )(features, labels, centers)
    return _LAMBDA_C * jnp.sum(partials) / 2.0 / _BATCH
